# baseline (speedup 1.0000x reference)
import functools

import jax
import jax.numpy as jnp
from jax import lax
from jax.experimental import pallas as pl
from jax.experimental.pallas import tpu as pltpu
from jax.experimental.pallas import tpu_sc as plsc

NC = 2
NS = 16
LANES = 16
CHUNK = 64
STRIP = 256
NSTRIP = 4
NGROUP = 8
NSPAN = 8
VPS = STRIP // LANES


def _span_sum_body(S, seq_hbm, bounds_hbm, part_hbm,
                   bounds_v, buf0, buf1, out_v, sem0, sem1):
    wid = lax.axis_index("s") * NC + lax.axis_index("c")
    g = wid // NSTRIP
    c0 = (wid % NSTRIP) * STRIP

    pltpu.sync_copy(bounds_hbm, bounds_v)
    bv = bounds_v[...]

    def accumulate_range(lo, hi, b):
        a0 = (lo // 8) * 8
        nchunks = jnp.where(lo < hi, (hi - a0 + CHUNK - 1) // CHUNK, 0)

        def dma_r0(k):
            return jnp.minimum(a0 + k * CHUNK, S - CHUNK)

        def src(k):
            return seq_hbm.at[b, pl.ds(dma_r0(k), CHUNK), pl.ds(c0, STRIP)]

        def issue(k, buf, sem):
            @pl.when(k < nchunks)
            def _():
                pltpu.async_copy(src(k), buf, sem)

        def drain(k, buf, sem):
            @pl.when(k < nchunks)
            def _():
                pltpu.make_async_copy(src(k), buf, sem).wait()

        def acc_chunk(k, buf, acc):
            r0 = a0 + k * CHUNK
            base = dma_r0(k)
            jlo = jnp.maximum(lo, r0) - base
            jhi = jnp.minimum(hi, r0 + CHUNK) - base
            cnt = jnp.maximum(jhi - jlo, 0)
            R = 4

            def rows4_body(i, acc):
                j = jlo + i * R
                for r in range(R):
                    acc = tuple(acc[h] + buf[j + r, pl.ds(h * LANES, LANES)]
                                for h in range(VPS))
                return acc

            acc = lax.fori_loop(0, cnt // R, rows4_body, acc)

            def row_body(j, acc):
                return tuple(acc[h] + buf[j, pl.ds(h * LANES, LANES)]
                             for h in range(VPS))

            return lax.fori_loop(jlo + (cnt // R) * R, jhi, row_body, acc)

        issue(0, buf0, sem0)

        def pair_body(k2, acc):
            a = 2 * k2
            issue(a + 1, buf1, sem1)
            drain(a, buf0, sem0)
            acc = acc_chunk(a, buf0, acc)
            issue(a + 2, buf0, sem0)
            drain(a + 1, buf1, sem1)
            return acc_chunk(a + 1, buf1, acc)

        acc0 = tuple(jnp.zeros((LANES,), jnp.float32) for _ in range(VPS))
        return lax.fori_loop(0, (nchunks + 1) // 2, pair_body, acc0)

    for s in range(NSPAN):
        s0 = bv[s]
        e0 = bv[s + NSPAN]
        n = e0 - s0 + 1
        q = (n + NGROUP - 1) // NGROUP
        lo = jnp.minimum(s0 + g * q, e0 + 1)
        hi = jnp.minimum(e0 + 1, lo + q)
        acc = accumulate_range(lo, hi, s // 2)
        for h in range(VPS):
            out_v[pl.ds(h * LANES, LANES)] = acc[h]
        pltpu.sync_copy(
            out_v, part_hbm.at[pl.ds((g * NSPAN + s) * 1024 + c0, STRIP)])


def kernel(sequence_output, entity_positions):
    B, S, H = sequence_output.shape
    pos = entity_positions
    h_start = jnp.clip(pos[:, 0], 0, S - 1)
    h_end = jnp.maximum(h_start, jnp.minimum(pos[:, 1], S - 1))
    t_start = jnp.clip(pos[:, 2], 0, S - 1)
    t_end = jnp.maximum(t_start, jnp.minimum(pos[:, 3], S - 1))
    starts = jnp.stack([h_start, t_start], axis=1).reshape(-1)
    ends = jnp.stack([h_end, t_end], axis=1).reshape(-1)
    bounds = jnp.concatenate([starts, ends]).astype(jnp.int32)

    mesh = plsc.VectorSubcoreMesh(
        core_axis_name="c", subcore_axis_name="s",
        num_cores=NC, num_subcores=NS)
    fn = pl.kernel(
        functools.partial(_span_sum_body, S),
        out_type=jax.ShapeDtypeStruct((NGROUP * NSPAN * H,), jnp.float32),
        mesh=mesh,
        compiler_params=pltpu.CompilerParams(needs_layout_passes=False),
        scratch_types=[
            pltpu.VMEM((16,), jnp.int32),
            pltpu.VMEM((CHUNK, STRIP), jnp.float32),
            pltpu.VMEM((CHUNK, STRIP), jnp.float32),
            pltpu.VMEM((STRIP,), jnp.float32),
            pltpu.SemaphoreType.DMA,
            pltpu.SemaphoreType.DMA,
        ],
    )
    partials = fn(sequence_output, bounds)
    sums = partials.reshape(NGROUP, NSPAN, H).sum(axis=0)
    counts = (ends - starts + 1).astype(jnp.float32)
    means = sums / counts[:, None]
    return means[0::2], means[1::2]


# speedup vs baseline: 1.1285x; 1.1285x over previous
import functools

import jax
import jax.numpy as jnp
from jax import lax
from jax.experimental import pallas as pl
from jax.experimental.pallas import tpu as pltpu
from jax.experimental.pallas import tpu_sc as plsc

NC = 2
NS = 16
LANES = 16
CHUNK = 64
STRIP = 256
NSTRIP = 4
NGROUP = 8
NSPAN = 8
VPS = STRIP // LANES


def _span_sum_body(S, seq_hbm, bounds_hbm, part_hbm,
                   bounds_v, buf0, buf1, out_v, sem0, sem1):
    wid = lax.axis_index("s") * NC + lax.axis_index("c")
    g = wid // NSTRIP
    c0 = (wid % NSTRIP) * STRIP

    pltpu.sync_copy(bounds_hbm, bounds_v)
    bv = bounds_v[...]

    def accumulate_range(lo, hi, b):
        a0 = (lo // 8) * 8
        nchunks = jnp.where(lo < hi, (hi - a0 + CHUNK - 1) // CHUNK, 0)

        def dma_r0(k):
            return jnp.minimum(a0 + k * CHUNK, S - CHUNK)

        def src(k):
            return seq_hbm.at[b, pl.ds(dma_r0(k), CHUNK), pl.ds(c0, STRIP)]

        def issue(k, buf, sem):
            @pl.when(k < nchunks)
            def _():
                pltpu.async_copy(src(k), buf, sem)

        def drain(k, buf, sem):
            @pl.when(k < nchunks)
            def _():
                pltpu.make_async_copy(src(k), buf, sem).wait()

        def acc_chunk(k, buf, acc):
            r0 = a0 + k * CHUNK
            base = dma_r0(k)
            jlo = jnp.maximum(lo, r0) - base
            jhi = jnp.minimum(hi, r0 + CHUNK) - base
            def row_body(j, acc):
                return tuple(acc[h] + buf[j, pl.ds(h * LANES, LANES)]
                             for h in range(VPS))

            return lax.fori_loop(jlo, jhi, row_body, acc)

        issue(0, buf0, sem0)

        def pair_body(k2, acc):
            a = 2 * k2
            issue(a + 1, buf1, sem1)
            drain(a, buf0, sem0)
            acc = acc_chunk(a, buf0, acc)
            issue(a + 2, buf0, sem0)
            drain(a + 1, buf1, sem1)
            return acc_chunk(a + 1, buf1, acc)

        acc0 = tuple(jnp.zeros((LANES,), jnp.float32) for _ in range(VPS))
        return lax.fori_loop(0, (nchunks + 1) // 2, pair_body, acc0)

    for s in range(NSPAN):
        s0 = bv[s]
        e0 = bv[s + NSPAN]
        n = e0 - s0 + 1
        q = (n + NGROUP - 1) // NGROUP
        lo = jnp.minimum(s0 + g * q, e0 + 1)
        hi = jnp.minimum(e0 + 1, lo + q)
        acc = accumulate_range(lo, hi, s // 2)
        for h in range(VPS):
            out_v[pl.ds(h * LANES, LANES)] = acc[h]
        pltpu.sync_copy(
            out_v, part_hbm.at[pl.ds((g * NSPAN + s) * 1024 + c0, STRIP)])


def kernel(sequence_output, entity_positions):
    B, S, H = sequence_output.shape
    pos = entity_positions
    h_start = jnp.clip(pos[:, 0], 0, S - 1)
    h_end = jnp.maximum(h_start, jnp.minimum(pos[:, 1], S - 1))
    t_start = jnp.clip(pos[:, 2], 0, S - 1)
    t_end = jnp.maximum(t_start, jnp.minimum(pos[:, 3], S - 1))
    starts = jnp.stack([h_start, t_start], axis=1).reshape(-1)
    ends = jnp.stack([h_end, t_end], axis=1).reshape(-1)
    bounds = jnp.concatenate([starts, ends]).astype(jnp.int32)

    mesh = plsc.VectorSubcoreMesh(
        core_axis_name="c", subcore_axis_name="s",
        num_cores=NC, num_subcores=NS)
    fn = pl.kernel(
        functools.partial(_span_sum_body, S),
        out_type=jax.ShapeDtypeStruct((NGROUP * NSPAN * H,), jnp.float32),
        mesh=mesh,
        compiler_params=pltpu.CompilerParams(needs_layout_passes=False),
        scratch_types=[
            pltpu.VMEM((16,), jnp.int32),
            pltpu.VMEM((CHUNK, STRIP), jnp.float32),
            pltpu.VMEM((CHUNK, STRIP), jnp.float32),
            pltpu.VMEM((STRIP,), jnp.float32),
            pltpu.SemaphoreType.DMA,
            pltpu.SemaphoreType.DMA,
        ],
    )
    partials = fn(sequence_output, bounds)
    sums = partials.reshape(NGROUP, NSPAN, H).sum(axis=0)
    counts = (ends - starts + 1).astype(jnp.float32)
    means = sums / counts[:, None]
    return means[0::2], means[1::2]
